# baseline (speedup 1.0000x reference)
import functools

import jax
import jax.numpy as jnp
from jax import lax
from jax.experimental import pallas as pl
from jax.experimental.pallas import tpu as pltpu
from jax.experimental.pallas import tpu_sc as plsc

N = 10000
E = 320000
D = 128
H = 32
OUT = 2

NC = 2
NS = 16
LANES = 16
NW = NC * NS

BATCH = 128
CH = 80
NBUF = 8
EDGES_PER_TILE = CH * BATCH
E_PAD = NW * EDGES_PER_TILE
N_PAD = 10240
RPT = N_PAD // NS
RCH = RPT // BATCH

_mesh = plsc.VectorSubcoreMesh(
    core_axis_name="c", subcore_axis_name="s", num_cores=NC, num_subcores=NS
)
_sc_params = pltpu.CompilerParams(
    needs_layout_passes=False, use_tc_tiling_on_sc=False
)


def _rsqrt16(d):
    i = plsc.bitcast(d, jnp.int32)
    i = jnp.int32(0x5F3759DF) - (i >> 1)
    y = plsc.bitcast(i, jnp.float32)
    for _ in range(3):
        y = y * (1.5 - 0.5 * d * y * y)
    return y


def _count_slice(dstbuf, degloc):
    ones16 = jnp.ones((LANES,), jnp.float32)

    def _count(i, _):
        for j in range(BATCH // LANES):
            idx = dstbuf[i, pl.ds(j * LANES, LANES)]
            plsc.addupdate_scatter(degloc, [idx], ones16)
        return 0

    lax.fori_loop(0, CH, _count, 0)


def _full_degree(dst_hbm, s, dstbuf, degloc, degsh, disbuf, dsems):
    zeros16 = jnp.zeros((LANES,), jnp.float32)

    def _zero(i, _):
        degloc[pl.ds(i * LANES, LANES)] = zeros16
        return 0

    lax.fori_loop(0, N_PAD // LANES, _zero, 0)
    pltpu.async_copy(degloc.at[pl.ds(0, RPT)], degsh.at[pl.ds(s * RPT, RPT)],
                     dsems[0])

    pltpu.sync_copy(dst_hbm.at[s], dstbuf)
    _count_slice(dstbuf, degloc)
    pltpu.sync_copy(dst_hbm.at[s + NS], dstbuf)
    _count_slice(dstbuf, degloc)

    iota16 = lax.iota(jnp.int32, LANES)

    def _iota(k, _):
        for j in range(BATCH // LANES):
            dstbuf[k, pl.ds(j * LANES, LANES)] = k * BATCH + j * LANES + iota16
        return 0

    lax.fori_loop(0, N_PAD // BATCH, _iota, 0)
    pltpu.make_async_copy(degloc.at[pl.ds(0, RPT)],
                          degsh.at[pl.ds(s * RPT, RPT)], dsems[0]).wait()
    plsc.subcore_barrier()

    nd = len(dsems)
    for b in range(nd):
        pltpu.async_copy(degloc.at[pl.ds(b * BATCH, BATCH)],
                         degsh.at[dstbuf.at[b]], dsems[b], add=True)

    def _red(it, _):
        k0 = it * nd
        for b in range(nd):
            pltpu.make_async_copy(degloc.at[pl.ds((k0 + b) * BATCH, BATCH)],
                                  degsh.at[dstbuf.at[k0 + b]], dsems[b]).wait()
            pltpu.async_copy(degloc.at[pl.ds((k0 + nd + b) * BATCH, BATCH)],
                             degsh.at[dstbuf.at[k0 + nd + b]], dsems[b],
                             add=True)
        return 0

    lax.fori_loop(0, N_PAD // BATCH // nd - 1, _red, 0)
    k0 = N_PAD // BATCH - nd
    for b in range(nd):
        pltpu.make_async_copy(degloc.at[pl.ds((k0 + b) * BATCH, BATCH)],
                              degsh.at[dstbuf.at[k0 + b]], dsems[b]).wait()
    plsc.subcore_barrier()

    pltpu.sync_copy(degsh.at[pl.ds(s * RPT, RPT)], degloc.at[pl.ds(0, RPT)])

    def _fin(j, _):
        acc = degloc[pl.ds(j * LANES, LANES)]
        disbuf[pl.ds(j * LANES, LANES)] = _rsqrt16(acc + 1.0)
        return 0

    lax.fori_loop(0, RPT // LANES, _fin, 0)


def _agg_pipeline(srcbuf, dstbuf, gtab, acc, rows, gsems, ssems):
    for b in range(NBUF):
        pltpu.async_copy(gtab.at[srcbuf.at[b]], rows.at[b], gsems[b])

    def _round(it, _):
        k0 = it * NBUF
        for b in range(NBUF):
            pltpu.make_async_copy(gtab.at[srcbuf.at[k0 + b]], rows.at[b],
                                  gsems[b]).wait()
            pltpu.async_copy(rows.at[b], acc.at[dstbuf.at[k0 + b]], ssems[b],
                             add=True)
        for b in range(NBUF):
            pltpu.make_async_copy(rows.at[b], acc.at[dstbuf.at[k0 + b]],
                                  ssems[b]).wait()
            pltpu.async_copy(gtab.at[srcbuf.at[k0 + NBUF + b]], rows.at[b],
                             gsems[b])
        return 0

    lax.fori_loop(0, CH // NBUF - 1, _round, 0)
    k0 = CH - NBUF
    for b in range(NBUF):
        pltpu.make_async_copy(gtab.at[srcbuf.at[k0 + b]], rows.at[b],
                              gsems[b]).wait()
        pltpu.async_copy(rows.at[b], acc.at[dstbuf.at[k0 + b]], ssems[b],
                         add=True)
    for b in range(NBUF):
        pltpu.make_async_copy(rows.at[b], acc.at[dstbuf.at[k0 + b]],
                              ssems[b]).wait()


def _zero_acc(s, zbuf, acc):
    zeros16 = jnp.zeros((LANES,), jnp.float32)

    def _zero(i, _):
        zbuf[i, pl.ds(0, LANES)] = zeros16
        zbuf[i, pl.ds(LANES, LANES)] = zeros16
        return 0

    lax.fori_loop(0, RPT, _zero, 0)
    pltpu.sync_copy(zbuf, acc.at[pl.ds(s * RPT, RPT)])


def _scale_rows(inbuf, outbuf, dis16):
    for r in range(LANES):
        bc = jnp.take(dis16, jnp.full((LANES,), r, jnp.int32))
        outbuf[r, pl.ds(0, LANES)] = inbuf[r, pl.ds(0, LANES)] * bc
        outbuf[r, pl.ds(LANES, LANES)] = inbuf[r, pl.ds(LANES, LANES)] * bc


_AGG1_SCRATCH = [
    pltpu.VMEM((CH, BATCH), jnp.int32),
    pltpu.VMEM((CH, BATCH), jnp.int32),
    pltpu.VMEM((N_PAD,), jnp.float32),
    pltpu.VMEM((RPT,), jnp.float32),
    pltpu.VMEM((BATCH, H), jnp.float32),
    pltpu.VMEM((NBUF, BATCH, H), jnp.float32),
    pltpu.VMEM((RPT, H), jnp.float32),
    pltpu.VMEM_SHARED((N_PAD,), jnp.float32),
    pltpu.VMEM_SHARED((N_PAD, H), jnp.float32),
    pltpu.VMEM_SHARED((N_PAD, H), jnp.float32),
    [pltpu.SemaphoreType.DMA] * NBUF,
    [pltpu.SemaphoreType.DMA] * NBUF,
]


@functools.partial(
    pl.kernel,
    out_type=(
        jax.ShapeDtypeStruct((NC, N_PAD, H), jnp.float32),
        jax.ShapeDtypeStruct((NC, N_PAD), jnp.float32),
    ),
    mesh=_mesh,
    compiler_params=_sc_params,
    scratch_types=_AGG1_SCRATCH,
)
def _agg1_kernel(h1_hbm, src_hbm, dst_hbm, s1_hbm, deg_hbm,
                 srcbuf, dstbuf, degloc, disbuf, inrows, rows, zbuf,
                 degsh, gtab, acc, gsems, ssems):
    c = lax.axis_index("c")
    s = lax.axis_index("s")
    gid = c * NS + s

    _full_degree(dst_hbm, s, dstbuf, degloc, degsh, disbuf, gsems[:4])
    pltpu.sync_copy(degloc.at[pl.ds(0, RPT)], deg_hbm.at[c, pl.ds(s * RPT, RPT)])

    def _build(k, _):
        base = k * BATCH
        pltpu.sync_copy(h1_hbm.at[pl.ds(s * RPT + base, BATCH)], inrows)
        for j in range(BATCH // LANES):
            dis16 = disbuf[pl.ds(base + j * LANES, LANES)]
            _scale_rows(inrows.at[pl.ds(j * LANES, LANES)],
                        zbuf.at[pl.ds(j * LANES, LANES)], dis16)
        pltpu.sync_copy(zbuf.at[pl.ds(0, BATCH)],
                        gtab.at[pl.ds(s * RPT + base, BATCH)])
        return 0

    lax.fori_loop(0, RCH, _build, 0)

    _zero_acc(s, zbuf, acc)
    plsc.subcore_barrier()

    pltpu.sync_copy(src_hbm.at[gid], srcbuf)
    pltpu.sync_copy(dst_hbm.at[gid], dstbuf)
    _agg_pipeline(srcbuf, dstbuf, gtab, acc, rows, gsems, ssems)

    plsc.subcore_barrier()
    pltpu.sync_copy(acc.at[pl.ds(s * RPT, RPT)], s1_hbm.at[c, pl.ds(s * RPT, RPT)])


_AGG2_SCRATCH = [
    pltpu.VMEM((CH, BATCH), jnp.int32),
    pltpu.VMEM((CH, BATCH), jnp.int32),
    pltpu.VMEM((RPT,), jnp.float32),
    pltpu.VMEM((H,), jnp.float32),
    pltpu.VMEM((BATCH, H), jnp.float32),
    pltpu.VMEM((BATCH, H), jnp.float32),
    pltpu.VMEM((BATCH, H), jnp.float32),
    pltpu.VMEM((NBUF, BATCH, H), jnp.float32),
    pltpu.VMEM((RPT, H), jnp.float32),
    pltpu.VMEM_SHARED((N_PAD, H), jnp.float32),
    pltpu.VMEM_SHARED((N_PAD, H), jnp.float32),
    [pltpu.SemaphoreType.DMA] * NBUF,
    [pltpu.SemaphoreType.DMA] * NBUF,
]


@functools.partial(
    pl.kernel,
    out_type=(
        jax.ShapeDtypeStruct((NC, N_PAD, H), jnp.float32),
        jax.ShapeDtypeStruct((N_PAD, H), jnp.float32),
    ),
    mesh=_mesh,
    compiler_params=_sc_params,
    scratch_types=_AGG2_SCRATCH,
)
def _agg2_kernel(s1_hbm, h1_hbm, deg_hbm, b1_hbm, src_hbm, dst_hbm,
                 r_hbm, q_hbm,
                 srcbuf, dstbuf, disbuf, bbuf, p0, p1, grows, rows, zbuf,
                 qtab, acc, gsems, ssems):
    c = lax.axis_index("c")
    s = lax.axis_index("s")
    gid = c * NS + s

    pltpu.sync_copy(b1_hbm, bbuf)
    b_lo = bbuf[pl.ds(0, LANES)]
    b_hi = bbuf[pl.ds(LANES, LANES)]

    pltpu.sync_copy(deg_hbm.at[c, pl.ds(s * RPT, RPT)], disbuf)

    def _newton(j, _):
        d = disbuf[pl.ds(j * LANES, LANES)]
        disbuf[pl.ds(j * LANES, LANES)] = _rsqrt16(d + 1.0)
        return 0

    lax.fori_loop(0, RPT // LANES, _newton, 0)

    def _build(k, _):
        base = s * RPT + k * BATCH
        pltpu.sync_copy(s1_hbm.at[0, pl.ds(base, BATCH)], p0)
        pltpu.sync_copy(s1_hbm.at[1, pl.ds(base, BATCH)], p1)
        pltpu.sync_copy(h1_hbm.at[pl.ds(base, BATCH)], grows)
        for j in range(BATCH // LANES):
            dis16 = disbuf[pl.ds(k * BATCH + j * LANES, LANES)]
            for r in range(LANES):
                row = j * LANES + r
                bc = jnp.take(dis16, jnp.full((LANES,), r, jnp.int32))
                lo = (p0[row, pl.ds(0, LANES)] + p1[row, pl.ds(0, LANES)]
                      + bc * grows[row, pl.ds(0, LANES)])
                hi = (p0[row, pl.ds(LANES, LANES)] + p1[row, pl.ds(LANES, LANES)]
                      + bc * grows[row, pl.ds(LANES, LANES)])
                lo = jnp.maximum(bc * lo + b_lo, 0.0) * bc
                hi = jnp.maximum(bc * hi + b_hi, 0.0) * bc
                zbuf[row, pl.ds(0, LANES)] = lo
                zbuf[row, pl.ds(LANES, LANES)] = hi
        pltpu.sync_copy(zbuf.at[pl.ds(0, BATCH)], qtab.at[pl.ds(base, BATCH)])
        return 0

    lax.fori_loop(0, RCH, _build, 0)

    @pl.when(c == 0)
    def _():
        pltpu.sync_copy(qtab.at[pl.ds(s * RPT, RPT)],
                        q_hbm.at[pl.ds(s * RPT, RPT)])

    _zero_acc(s, zbuf, acc)
    plsc.subcore_barrier()

    pltpu.sync_copy(src_hbm.at[gid], srcbuf)
    pltpu.sync_copy(dst_hbm.at[gid], dstbuf)
    _agg_pipeline(srcbuf, dstbuf, qtab, acc, rows, gsems, ssems)

    plsc.subcore_barrier()
    pltpu.sync_copy(acc.at[pl.ds(s * RPT, RPT)], r_hbm.at[c, pl.ds(s * RPT, RPT)])


def _tc_a_body(x_ref, w1_ref, h_ref):
    h_ref[:N, :] = jnp.dot(x_ref[...], w1_ref[...],
                           preferred_element_type=jnp.float32)


def _tc_c_body(degp_ref, q_ref, r_ref, w2_ref, b2_ref, o_ref):
    d = (degp_ref[:, 0:1] + degp_ref[:, 1:2]) * 0.5 + 1.0
    dis = lax.rsqrt(d)
    rsum = r_ref[0, :N, :] + r_ref[1, :N, :]
    t = dis * (rsum + q_ref[:N, :])
    o = jnp.dot(t, w2_ref[...], preferred_element_type=jnp.float32) + b2_ref[...]
    a = o[:, 0:1]
    b = o[:, 1:2]
    m = jnp.maximum(a, b)
    lse = m + jnp.log(jnp.exp(a - m) + jnp.exp(b - m))
    o_ref[...] = o - lse


def _vmem_call(body, n_in, out_shape):
    return pl.pallas_call(
        body,
        out_shape=out_shape,
        in_specs=[pl.BlockSpec(memory_space=pltpu.VMEM)] * n_in,
        out_specs=pl.BlockSpec(memory_space=pltpu.VMEM),
    )


@jax.jit
def kernel(x, edge_index, W1, b1, W2, b2):
    src = edge_index[0].astype(jnp.int32)
    dst = edge_index[1].astype(jnp.int32)
    pad_iota = jnp.arange(E_PAD - E, dtype=jnp.int32)
    src_p = jnp.concatenate([src, pad_iota % N])
    dst_p = jnp.concatenate([dst, N + pad_iota % (N_PAD - N)])
    src_t = src_p.reshape(NW, CH, BATCH)
    dst_t = dst_p.reshape(NW, CH, BATCH)

    h1 = _vmem_call(
        _tc_a_body, 2, jax.ShapeDtypeStruct((N_PAD, H), jnp.float32)
    )(x, W1)

    s1, deg = _agg1_kernel(h1, src_t, dst_t)
    r, q = _agg2_kernel(s1, h1, deg, b1, src_t, dst_t)

    degp_t = deg.T[:N, :]
    out = _vmem_call(
        _tc_c_body, 5, jax.ShapeDtypeStruct((N, OUT), jnp.float32)
    )(degp_t, q, r, W2, b2.reshape(1, OUT))
    return out


# speedup vs baseline: 1.0061x; 1.0061x over previous
import functools

import jax
import jax.numpy as jnp
from jax import lax
from jax.experimental import pallas as pl
from jax.experimental.pallas import tpu as pltpu
from jax.experimental.pallas import tpu_sc as plsc

N = 10000
E = 320000
D = 128
H = 32
OUT = 2

NC = 2
NS = 16
LANES = 16
NW = NC * NS

BATCH = 128
CH = 80
NBUF = 8
EDGES_PER_TILE = CH * BATCH
E_PAD = NW * EDGES_PER_TILE
N_PAD = 10240
RPT = N_PAD // NS
RCH = RPT // BATCH

_mesh = plsc.VectorSubcoreMesh(
    core_axis_name="c", subcore_axis_name="s", num_cores=NC, num_subcores=NS
)
_sc_params = pltpu.CompilerParams(
    needs_layout_passes=False, use_tc_tiling_on_sc=False
)


def _rsqrt16(d):
    i = plsc.bitcast(d, jnp.int32)
    i = jnp.int32(0x5F3759DF) - (i >> 1)
    y = plsc.bitcast(i, jnp.float32)
    for _ in range(3):
        y = y * (1.5 - 0.5 * d * y * y)
    return y


def _count_slice(dstbuf, degloc):
    ones16 = jnp.ones((LANES,), jnp.float32)

    def _count(i, _):
        for j in range(BATCH // LANES):
            idx = dstbuf[i, pl.ds(j * LANES, LANES)]
            plsc.addupdate_scatter(degloc, [idx], ones16)
        return 0

    lax.fori_loop(0, CH, _count, 0)


def _full_degree(dst_hbm, s, dstbuf, degloc, degsh, disbuf, dsems):
    zeros16 = jnp.zeros((LANES,), jnp.float32)

    def _zero(i, _):
        degloc[pl.ds(i * LANES, LANES)] = zeros16
        return 0

    lax.fori_loop(0, N_PAD // LANES, _zero, 0)
    pltpu.async_copy(degloc.at[pl.ds(0, RPT)], degsh.at[pl.ds(s * RPT, RPT)],
                     dsems[0])

    pltpu.sync_copy(dst_hbm.at[s], dstbuf)
    _count_slice(dstbuf, degloc)
    pltpu.sync_copy(dst_hbm.at[s + NS], dstbuf)
    _count_slice(dstbuf, degloc)

    iota16 = lax.iota(jnp.int32, LANES)

    def _iota(k, _):
        for j in range(BATCH // LANES):
            dstbuf[k, pl.ds(j * LANES, LANES)] = k * BATCH + j * LANES + iota16
        return 0

    lax.fori_loop(0, N_PAD // BATCH, _iota, 0)
    pltpu.make_async_copy(degloc.at[pl.ds(0, RPT)],
                          degsh.at[pl.ds(s * RPT, RPT)], dsems[0]).wait()
    plsc.subcore_barrier()

    nd = len(dsems)
    for b in range(nd):
        pltpu.async_copy(degloc.at[pl.ds(b * BATCH, BATCH)],
                         degsh.at[dstbuf.at[b]], dsems[b], add=True)

    def _red(it, _):
        k0 = it * nd
        for b in range(nd):
            pltpu.make_async_copy(degloc.at[pl.ds((k0 + b) * BATCH, BATCH)],
                                  degsh.at[dstbuf.at[k0 + b]], dsems[b]).wait()
            pltpu.async_copy(degloc.at[pl.ds((k0 + nd + b) * BATCH, BATCH)],
                             degsh.at[dstbuf.at[k0 + nd + b]], dsems[b],
                             add=True)
        return 0

    lax.fori_loop(0, N_PAD // BATCH // nd - 1, _red, 0)
    k0 = N_PAD // BATCH - nd
    for b in range(nd):
        pltpu.make_async_copy(degloc.at[pl.ds((k0 + b) * BATCH, BATCH)],
                              degsh.at[dstbuf.at[k0 + b]], dsems[b]).wait()
    plsc.subcore_barrier()

    pltpu.sync_copy(degsh.at[pl.ds(s * RPT, RPT)], degloc.at[pl.ds(0, RPT)])

    def _fin(j, _):
        acc = degloc[pl.ds(j * LANES, LANES)]
        disbuf[pl.ds(j * LANES, LANES)] = _rsqrt16(acc + 1.0)
        return 0

    lax.fori_loop(0, RPT // LANES, _fin, 0)


def _agg_pipeline(srcbuf, dstbuf, gtab, acc, rows, gsems, ssems):
    for b in range(NBUF):
        pltpu.async_copy(gtab.at[srcbuf.at[b]], rows.at[b], gsems[b])

    def _round(it, _):
        k0 = it * NBUF
        for b in range(NBUF):
            pltpu.make_async_copy(gtab.at[srcbuf.at[k0 + b]], rows.at[b],
                                  gsems[b]).wait()
            pltpu.async_copy(rows.at[b], acc.at[dstbuf.at[k0 + b]], ssems[b],
                             add=True)
        for b in range(NBUF):
            pltpu.make_async_copy(rows.at[b], acc.at[dstbuf.at[k0 + b]],
                                  ssems[b]).wait()
            pltpu.async_copy(gtab.at[srcbuf.at[k0 + NBUF + b]], rows.at[b],
                             gsems[b])
        return 0

    lax.fori_loop(0, CH // NBUF - 1, _round, 0)
    k0 = CH - NBUF
    for b in range(NBUF):
        pltpu.make_async_copy(gtab.at[srcbuf.at[k0 + b]], rows.at[b],
                              gsems[b]).wait()
        pltpu.async_copy(rows.at[b], acc.at[dstbuf.at[k0 + b]], ssems[b],
                         add=True)
    for b in range(NBUF):
        pltpu.make_async_copy(rows.at[b], acc.at[dstbuf.at[k0 + b]],
                              ssems[b]).wait()


def _zero_acc(s, zbuf, acc):
    zeros16 = jnp.zeros((LANES,), jnp.float32)

    def _zero(i, _):
        zbuf[i, pl.ds(0, LANES)] = zeros16
        zbuf[i, pl.ds(LANES, LANES)] = zeros16
        return 0

    lax.fori_loop(0, RPT, _zero, 0)
    pltpu.sync_copy(zbuf, acc.at[pl.ds(s * RPT, RPT)])


def _scale_rows(inbuf, outbuf, dis16):
    for r in range(LANES):
        bc = jnp.take(dis16, jnp.full((LANES,), r, jnp.int32))
        outbuf[r, pl.ds(0, LANES)] = inbuf[r, pl.ds(0, LANES)] * bc
        outbuf[r, pl.ds(LANES, LANES)] = inbuf[r, pl.ds(LANES, LANES)] * bc


_AGG1_SCRATCH = [
    pltpu.VMEM((CH, BATCH), jnp.int32),
    pltpu.VMEM((CH, BATCH), jnp.int32),
    pltpu.VMEM((N_PAD,), jnp.float32),
    pltpu.VMEM((RPT,), jnp.float32),
    pltpu.VMEM((BATCH, H), jnp.float32),
    pltpu.VMEM((NBUF, BATCH, H), jnp.float32),
    pltpu.VMEM((RPT, H), jnp.float32),
    pltpu.VMEM_SHARED((N_PAD,), jnp.float32),
    pltpu.VMEM_SHARED((N_PAD, H), jnp.float32),
    pltpu.VMEM_SHARED((N_PAD, H), jnp.float32),
    [pltpu.SemaphoreType.DMA] * NBUF,
    [pltpu.SemaphoreType.DMA] * NBUF,
]


@functools.partial(
    pl.kernel,
    out_type=(
        jax.ShapeDtypeStruct((NC, N_PAD, H), jnp.float32),
        jax.ShapeDtypeStruct((NC, N_PAD), jnp.float32),
    ),
    mesh=_mesh,
    compiler_params=_sc_params,
    scratch_types=_AGG1_SCRATCH,
)
def _agg1_kernel(h1_hbm, src_hbm, dst_hbm, s1_hbm, deg_hbm,
                 srcbuf, dstbuf, degloc, disbuf, inrows, rows, zbuf,
                 degsh, gtab, acc, gsems, ssems):
    c = lax.axis_index("c")
    s = lax.axis_index("s")
    gid = c * NS + s

    with jax.named_scope("ph_degree"):
        _full_degree(dst_hbm, s, dstbuf, degloc, degsh, disbuf, gsems[:4])
    pltpu.sync_copy(degloc.at[pl.ds(0, RPT)], deg_hbm.at[c, pl.ds(s * RPT, RPT)])

    def _build(k, _):
        base = k * BATCH
        pltpu.sync_copy(h1_hbm.at[pl.ds(s * RPT + base, BATCH)], inrows)
        for j in range(BATCH // LANES):
            dis16 = disbuf[pl.ds(base + j * LANES, LANES)]
            _scale_rows(inrows.at[pl.ds(j * LANES, LANES)],
                        zbuf.at[pl.ds(j * LANES, LANES)], dis16)
        pltpu.sync_copy(zbuf.at[pl.ds(0, BATCH)],
                        gtab.at[pl.ds(s * RPT + base, BATCH)])
        return 0

    with jax.named_scope("ph_build1"):
        lax.fori_loop(0, RCH, _build, 0)

    with jax.named_scope("ph_zero1"):
        _zero_acc(s, zbuf, acc)
    plsc.subcore_barrier()

    with jax.named_scope("ph_agg1"):
        pltpu.sync_copy(src_hbm.at[gid], srcbuf)
        pltpu.sync_copy(dst_hbm.at[gid], dstbuf)
        _agg_pipeline(srcbuf, dstbuf, gtab, acc, rows, gsems, ssems)

    plsc.subcore_barrier()
    pltpu.sync_copy(acc.at[pl.ds(s * RPT, RPT)], s1_hbm.at[c, pl.ds(s * RPT, RPT)])


_AGG2_SCRATCH = [
    pltpu.VMEM((CH, BATCH), jnp.int32),
    pltpu.VMEM((CH, BATCH), jnp.int32),
    pltpu.VMEM((RPT,), jnp.float32),
    pltpu.VMEM((H,), jnp.float32),
    pltpu.VMEM((BATCH, H), jnp.float32),
    pltpu.VMEM((BATCH, H), jnp.float32),
    pltpu.VMEM((BATCH, H), jnp.float32),
    pltpu.VMEM((NBUF, BATCH, H), jnp.float32),
    pltpu.VMEM((RPT, H), jnp.float32),
    pltpu.VMEM_SHARED((N_PAD, H), jnp.float32),
    pltpu.VMEM_SHARED((N_PAD, H), jnp.float32),
    [pltpu.SemaphoreType.DMA] * NBUF,
    [pltpu.SemaphoreType.DMA] * NBUF,
]


@functools.partial(
    pl.kernel,
    out_type=(
        jax.ShapeDtypeStruct((NC, N_PAD, H), jnp.float32),
        jax.ShapeDtypeStruct((N_PAD, H), jnp.float32),
    ),
    mesh=_mesh,
    compiler_params=_sc_params,
    scratch_types=_AGG2_SCRATCH,
)
def _agg2_kernel(s1_hbm, h1_hbm, deg_hbm, b1_hbm, src_hbm, dst_hbm,
                 r_hbm, q_hbm,
                 srcbuf, dstbuf, disbuf, bbuf, p0, p1, grows, rows, zbuf,
                 qtab, acc, gsems, ssems):
    c = lax.axis_index("c")
    s = lax.axis_index("s")
    gid = c * NS + s

    pltpu.sync_copy(b1_hbm, bbuf)
    b_lo = bbuf[pl.ds(0, LANES)]
    b_hi = bbuf[pl.ds(LANES, LANES)]

    pltpu.sync_copy(deg_hbm.at[c, pl.ds(s * RPT, RPT)], disbuf)

    def _newton(j, _):
        d = disbuf[pl.ds(j * LANES, LANES)]
        disbuf[pl.ds(j * LANES, LANES)] = _rsqrt16(d + 1.0)
        return 0

    lax.fori_loop(0, RPT // LANES, _newton, 0)

    def _build(k, _):
        base = s * RPT + k * BATCH
        pltpu.sync_copy(s1_hbm.at[0, pl.ds(base, BATCH)], p0)
        pltpu.sync_copy(s1_hbm.at[1, pl.ds(base, BATCH)], p1)
        pltpu.sync_copy(h1_hbm.at[pl.ds(base, BATCH)], grows)
        for j in range(BATCH // LANES):
            dis16 = disbuf[pl.ds(k * BATCH + j * LANES, LANES)]
            for r in range(LANES):
                row = j * LANES + r
                bc = jnp.take(dis16, jnp.full((LANES,), r, jnp.int32))
                lo = (p0[row, pl.ds(0, LANES)] + p1[row, pl.ds(0, LANES)]
                      + bc * grows[row, pl.ds(0, LANES)])
                hi = (p0[row, pl.ds(LANES, LANES)] + p1[row, pl.ds(LANES, LANES)]
                      + bc * grows[row, pl.ds(LANES, LANES)])
                lo = jnp.maximum(bc * lo + b_lo, 0.0) * bc
                hi = jnp.maximum(bc * hi + b_hi, 0.0) * bc
                zbuf[row, pl.ds(0, LANES)] = lo
                zbuf[row, pl.ds(LANES, LANES)] = hi
        pltpu.sync_copy(zbuf.at[pl.ds(0, BATCH)], qtab.at[pl.ds(base, BATCH)])
        return 0

    with jax.named_scope("ph_build2"):
        lax.fori_loop(0, RCH, _build, 0)

    with jax.named_scope("ph_qout"):
        @pl.when(c == 0)
        def _():
            pltpu.sync_copy(qtab.at[pl.ds(s * RPT, RPT)],
                            q_hbm.at[pl.ds(s * RPT, RPT)])

    with jax.named_scope("ph_zero2"):
        _zero_acc(s, zbuf, acc)
    plsc.subcore_barrier()

    with jax.named_scope("ph_agg2"):
        pltpu.sync_copy(src_hbm.at[gid], srcbuf)
        pltpu.sync_copy(dst_hbm.at[gid], dstbuf)
        _agg_pipeline(srcbuf, dstbuf, qtab, acc, rows, gsems, ssems)

    plsc.subcore_barrier()
    pltpu.sync_copy(acc.at[pl.ds(s * RPT, RPT)], r_hbm.at[c, pl.ds(s * RPT, RPT)])


def _tc_a_body(x_ref, w1_ref, h_ref):
    h_ref[:N, :] = jnp.dot(x_ref[...], w1_ref[...],
                           preferred_element_type=jnp.float32)


def _tc_c_body(degp_ref, q_ref, r_ref, w2_ref, b2_ref, o_ref):
    d = (degp_ref[:, 0:1] + degp_ref[:, 1:2]) * 0.5 + 1.0
    dis = lax.rsqrt(d)
    rsum = r_ref[0, :N, :] + r_ref[1, :N, :]
    t = dis * (rsum + q_ref[:N, :])
    o = jnp.dot(t, w2_ref[...], preferred_element_type=jnp.float32) + b2_ref[...]
    a = o[:, 0:1]
    b = o[:, 1:2]
    m = jnp.maximum(a, b)
    lse = m + jnp.log(jnp.exp(a - m) + jnp.exp(b - m))
    o_ref[...] = o - lse


def _vmem_call(body, n_in, out_shape):
    return pl.pallas_call(
        body,
        out_shape=out_shape,
        in_specs=[pl.BlockSpec(memory_space=pltpu.VMEM)] * n_in,
        out_specs=pl.BlockSpec(memory_space=pltpu.VMEM),
    )


@jax.jit
def kernel(x, edge_index, W1, b1, W2, b2):
    src = edge_index[0].astype(jnp.int32)
    dst = edge_index[1].astype(jnp.int32)
    pad_iota = jnp.arange(E_PAD - E, dtype=jnp.int32)
    src_p = jnp.concatenate([src, pad_iota % N])
    dst_p = jnp.concatenate([dst, N + pad_iota % (N_PAD - N)])
    src_t = src_p.reshape(NW, CH, BATCH)
    dst_t = dst_p.reshape(NW, CH, BATCH)

    h1 = _vmem_call(
        _tc_a_body, 2, jax.ShapeDtypeStruct((N_PAD, H), jnp.float32)
    )(x, W1)

    s1, deg = _agg1_kernel(h1, src_t, dst_t)
    r, q = _agg2_kernel(s1, h1, deg, b1, src_t, dst_t)

    degp_t = deg.T[:N, :]
    out = _vmem_call(
        _tc_c_body, 5, jax.ShapeDtypeStruct((N, OUT), jnp.float32)
    )(degp_t, q, r, W2, b2.reshape(1, OUT))
    return out


# speedup vs baseline: 1.0787x; 1.0722x over previous
import functools

import jax
import jax.numpy as jnp
from jax import lax
from jax.experimental import pallas as pl
from jax.experimental.pallas import tpu as pltpu
from jax.experimental.pallas import tpu_sc as plsc

N = 10000
E = 320000
D = 128
H = 32
OUT = 2

NC = 2
NS = 16
LANES = 16
NW = NC * NS

BATCH = 128
CH = 80
NBUF = 4
EDGES_PER_TILE = CH * BATCH
E_PAD = NW * EDGES_PER_TILE
N_PAD = 10240
RPT = N_PAD // NS

_mesh = plsc.VectorSubcoreMesh(
    core_axis_name="c", subcore_axis_name="s", num_cores=NC, num_subcores=NS
)
_sc_params = pltpu.CompilerParams(
    needs_layout_passes=False, use_tc_tiling_on_sc=False
)


@functools.partial(
    pl.kernel,
    out_type=jax.ShapeDtypeStruct((NC, N_PAD), jnp.float32),
    mesh=_mesh,
    compiler_params=_sc_params,
    scratch_types=[
        pltpu.VMEM((CH, BATCH), jnp.int32),
        pltpu.VMEM((N_PAD,), jnp.float32),
        pltpu.VMEM((NS, RPT), jnp.float32),
        pltpu.VMEM((RPT,), jnp.float32),
        pltpu.VMEM_SHARED((NS, N_PAD), jnp.float32),
    ],
)
def _deg_kernel(dst_hbm, out_hbm, dstbuf, degloc, stripe, outbuf, degsh):
    c = lax.axis_index("c")
    s = lax.axis_index("s")
    gid = c * NS + s

    pltpu.sync_copy(dst_hbm.at[gid], dstbuf)

    zeros16 = jnp.zeros((LANES,), jnp.float32)

    def _zero(i, _):
        degloc[pl.ds(i * LANES, LANES)] = zeros16
        return 0

    lax.fori_loop(0, N_PAD // LANES, _zero, 0)

    ones16 = jnp.ones((LANES,), jnp.float32)

    def _count(i, _):
        for j in range(BATCH // LANES):
            idx = dstbuf[i, pl.ds(j * LANES, LANES)]
            plsc.addupdate_scatter(degloc, [idx], ones16)
        return 0

    lax.fori_loop(0, CH, _count, 0)

    pltpu.sync_copy(degloc, degsh.at[s])
    plsc.subcore_barrier()
    pltpu.sync_copy(degsh.at[:, pl.ds(s * RPT, RPT)], stripe)

    def _reduce(j, _):
        acc = stripe[0, pl.ds(j * LANES, LANES)]
        for t in range(1, NS):
            acc = acc + stripe[t, pl.ds(j * LANES, LANES)]
        outbuf[pl.ds(j * LANES, LANES)] = acc
        return 0

    lax.fori_loop(0, RPT // LANES, _reduce, 0)
    pltpu.sync_copy(outbuf, out_hbm.at[c, pl.ds(s * RPT, RPT)])


@functools.partial(
    pl.kernel,
    out_type=jax.ShapeDtypeStruct((NC, N_PAD, H), jnp.float32),
    mesh=_mesh,
    compiler_params=_sc_params,
    scratch_types=[
        pltpu.VMEM((CH, BATCH), jnp.int32),
        pltpu.VMEM((CH, BATCH), jnp.int32),
        pltpu.VMEM((NBUF, BATCH, H), jnp.float32),
        pltpu.VMEM((RPT, H), jnp.float32),
        pltpu.VMEM_SHARED((N_PAD, H), jnp.float32),
        [pltpu.SemaphoreType.DMA] * NBUF,
        [pltpu.SemaphoreType.DMA] * NBUF,
    ],
)
def _agg_kernel(g_hbm, src_hbm, dst_hbm, out_hbm,
                srcbuf, dstbuf, rows, zbuf, acc, gsems, ssems):
    c = lax.axis_index("c")
    s = lax.axis_index("s")
    gid = c * NS + s

    pltpu.sync_copy(src_hbm.at[gid], srcbuf)
    pltpu.sync_copy(dst_hbm.at[gid], dstbuf)

    zeros16 = jnp.zeros((LANES,), jnp.float32)

    def _zero(i, _):
        zbuf[i, pl.ds(0, LANES)] = zeros16
        zbuf[i, pl.ds(LANES, LANES)] = zeros16
        return 0

    lax.fori_loop(0, RPT, _zero, 0)
    pltpu.sync_copy(zbuf, acc.at[pl.ds(s * RPT, RPT)])
    plsc.subcore_barrier()

    for b in range(NBUF):
        pltpu.async_copy(g_hbm.at[srcbuf.at[b]], rows.at[b], gsems[b])

    def _round(it, _):
        k0 = it * NBUF
        for b in range(NBUF):
            pltpu.make_async_copy(g_hbm.at[srcbuf.at[k0 + b]], rows.at[b],
                                  gsems[b]).wait()
            pltpu.async_copy(rows.at[b], acc.at[dstbuf.at[k0 + b]], ssems[b],
                             add=True)
        for b in range(NBUF):
            pltpu.make_async_copy(rows.at[b], acc.at[dstbuf.at[k0 + b]],
                                  ssems[b]).wait()
            pltpu.async_copy(g_hbm.at[srcbuf.at[k0 + NBUF + b]], rows.at[b],
                             gsems[b])
        return 0

    lax.fori_loop(0, CH // NBUF - 1, _round, 0)
    k0 = CH - NBUF
    for b in range(NBUF):
        pltpu.make_async_copy(g_hbm.at[srcbuf.at[k0 + b]], rows.at[b],
                              gsems[b]).wait()
        pltpu.async_copy(rows.at[b], acc.at[dstbuf.at[k0 + b]], ssems[b],
                         add=True)
    for b in range(NBUF):
        pltpu.make_async_copy(rows.at[b], acc.at[dstbuf.at[k0 + b]],
                              ssems[b]).wait()

    plsc.subcore_barrier()
    pltpu.sync_copy(acc.at[pl.ds(s * RPT, RPT)], out_hbm.at[c, pl.ds(s * RPT, RPT)])


def _dis_from(degp_ref):
    d = degp_ref[:, 0:1] + degp_ref[:, 1:2] + 1.0
    return lax.rsqrt(d)


def _tc_a_body(x_ref, w1_ref, degp_ref, g1_ref):
    dis = _dis_from(degp_ref)
    h = jnp.dot(x_ref[...], w1_ref[...], preferred_element_type=jnp.float32)
    g1_ref[...] = dis * h


def _tc_b_body(degp_ref, g1_ref, s1_ref, b1_ref, q_ref):
    dis = _dis_from(degp_ref)
    ssum = s1_ref[0, :N, :] + s1_ref[1, :N, :]
    out1 = dis * (ssum + g1_ref[...]) + b1_ref[...]
    q_ref[...] = dis * jnp.maximum(out1, 0.0)


def _tc_c_body(degp_ref, q_ref, r_ref, w2_ref, b2_ref, o_ref):
    dis = _dis_from(degp_ref)
    rsum = r_ref[0, :N, :] + r_ref[1, :N, :]
    t = dis * (rsum + q_ref[...])
    o = jnp.dot(t, w2_ref[...], preferred_element_type=jnp.float32) + b2_ref[...]
    a = o[:, 0:1]
    b = o[:, 1:2]
    m = jnp.maximum(a, b)
    lse = m + jnp.log(jnp.exp(a - m) + jnp.exp(b - m))
    o_ref[...] = o - lse


def _vmem_call(body, n_in, out_shape):
    return pl.pallas_call(
        body,
        out_shape=out_shape,
        in_specs=[pl.BlockSpec(memory_space=pltpu.VMEM)] * n_in,
        out_specs=pl.BlockSpec(memory_space=pltpu.VMEM),
    )


@jax.jit
def kernel(x, edge_index, W1, b1, W2, b2):
    src = edge_index[0].astype(jnp.int32)
    dst = edge_index[1].astype(jnp.int32)
    pad_iota = jnp.arange(E_PAD - E, dtype=jnp.int32)
    src_p = jnp.concatenate([src, pad_iota % N])
    dst_p = jnp.concatenate([dst, N + pad_iota % (N_PAD - N)])
    src_t = src_p.reshape(NW, CH, BATCH)
    dst_t = dst_p.reshape(NW, CH, BATCH)

    degp = _deg_kernel(dst_t)
    degp_t = degp.T[:N, :]

    g1 = _vmem_call(
        _tc_a_body, 3, jax.ShapeDtypeStruct((N, H), jnp.float32)
    )(x, W1, degp_t)

    s1 = _agg_kernel(g1, src_t, dst_t)

    q = _vmem_call(
        _tc_b_body, 4, jax.ShapeDtypeStruct((N, H), jnp.float32)
    )(degp_t, g1, s1, b1.reshape(1, H))

    r = _agg_kernel(q, src_t, dst_t)

    out = _vmem_call(
        _tc_c_body, 5, jax.ShapeDtypeStruct((N, OUT), jnp.float32)
    )(degp_t, q, r, W2, b2.reshape(1, OUT))
    return out


# speedup vs baseline: 1.1220x; 1.0402x over previous
import functools

import jax
import jax.numpy as jnp
from jax import lax
from jax.experimental import pallas as pl
from jax.experimental.pallas import tpu as pltpu
from jax.experimental.pallas import tpu_sc as plsc

N = 10000
E = 320000
D = 128
H = 32
OUT = 2

NC = 2
NS = 16
LANES = 16
NW = NC * NS

BATCH = 128
CH = 80
NBUF = 8
EDGES_PER_TILE = CH * BATCH
E_PAD = NW * EDGES_PER_TILE
N_PAD = 10240
RPT = N_PAD // NS

_mesh = plsc.VectorSubcoreMesh(
    core_axis_name="c", subcore_axis_name="s", num_cores=NC, num_subcores=NS
)
_sc_params = pltpu.CompilerParams(
    needs_layout_passes=False, use_tc_tiling_on_sc=False
)


@functools.partial(
    pl.kernel,
    out_type=jax.ShapeDtypeStruct((NC, N_PAD), jnp.float32),
    mesh=_mesh,
    compiler_params=_sc_params,
    scratch_types=[
        pltpu.VMEM((CH, BATCH), jnp.int32),
        pltpu.VMEM((N_PAD,), jnp.float32),
        pltpu.VMEM((NS, RPT), jnp.float32),
        pltpu.VMEM((RPT,), jnp.float32),
        pltpu.VMEM_SHARED((NS, N_PAD), jnp.float32),
    ],
)
def _deg_kernel(dst_hbm, out_hbm, dstbuf, degloc, stripe, outbuf, degsh):
    c = lax.axis_index("c")
    s = lax.axis_index("s")
    gid = c * NS + s

    pltpu.sync_copy(dst_hbm.at[gid], dstbuf)

    zeros16 = jnp.zeros((LANES,), jnp.float32)

    def _zero(i, _):
        degloc[pl.ds(i * LANES, LANES)] = zeros16
        return 0

    lax.fori_loop(0, N_PAD // LANES, _zero, 0)

    ones16 = jnp.ones((LANES,), jnp.float32)

    def _count(i, _):
        for j in range(BATCH // LANES):
            idx = dstbuf[i, pl.ds(j * LANES, LANES)]
            plsc.addupdate_scatter(degloc, [idx], ones16)
        return 0

    lax.fori_loop(0, CH, _count, 0)

    pltpu.sync_copy(degloc, degsh.at[s])
    plsc.subcore_barrier()
    pltpu.sync_copy(degsh.at[:, pl.ds(s * RPT, RPT)], stripe)

    def _reduce(j, _):
        acc = stripe[0, pl.ds(j * LANES, LANES)]
        for t in range(1, NS):
            acc = acc + stripe[t, pl.ds(j * LANES, LANES)]
        outbuf[pl.ds(j * LANES, LANES)] = acc
        return 0

    lax.fori_loop(0, RPT // LANES, _reduce, 0)
    pltpu.sync_copy(outbuf, out_hbm.at[c, pl.ds(s * RPT, RPT)])


@functools.partial(
    pl.kernel,
    out_type=jax.ShapeDtypeStruct((NC, N_PAD, H), jnp.float32),
    mesh=_mesh,
    compiler_params=_sc_params,
    scratch_types=[
        pltpu.VMEM((CH, BATCH), jnp.int32),
        pltpu.VMEM((CH, BATCH), jnp.int32),
        pltpu.VMEM((NBUF, BATCH, H), jnp.float32),
        pltpu.VMEM((RPT, H), jnp.float32),
        pltpu.VMEM_SHARED((N_PAD, H), jnp.float32),
        [pltpu.SemaphoreType.DMA] * NBUF,
        [pltpu.SemaphoreType.DMA] * NBUF,
    ],
)
def _agg_kernel(g_hbm, src_hbm, dst_hbm, out_hbm,
                srcbuf, dstbuf, rows, zbuf, acc, gsems, ssems):
    c = lax.axis_index("c")
    s = lax.axis_index("s")
    gid = c * NS + s

    pltpu.sync_copy(src_hbm.at[gid], srcbuf)
    pltpu.sync_copy(dst_hbm.at[gid], dstbuf)

    zeros16 = jnp.zeros((LANES,), jnp.float32)

    def _zero(i, _):
        zbuf[i, pl.ds(0, LANES)] = zeros16
        zbuf[i, pl.ds(LANES, LANES)] = zeros16
        return 0

    lax.fori_loop(0, RPT, _zero, 0)
    pltpu.sync_copy(zbuf, acc.at[pl.ds(s * RPT, RPT)])
    plsc.subcore_barrier()

    for b in range(NBUF):
        pltpu.async_copy(g_hbm.at[srcbuf.at[b]], rows.at[b], gsems[b])

    def _round(it, _):
        k0 = it * NBUF
        for b in range(NBUF):
            pltpu.make_async_copy(g_hbm.at[srcbuf.at[k0 + b]], rows.at[b],
                                  gsems[b]).wait()
            pltpu.async_copy(rows.at[b], acc.at[dstbuf.at[k0 + b]], ssems[b],
                             add=True)
        for b in range(NBUF):
            pltpu.make_async_copy(rows.at[b], acc.at[dstbuf.at[k0 + b]],
                                  ssems[b]).wait()
            pltpu.async_copy(g_hbm.at[srcbuf.at[k0 + NBUF + b]], rows.at[b],
                             gsems[b])
        return 0

    lax.fori_loop(0, CH // NBUF - 1, _round, 0)
    k0 = CH - NBUF
    for b in range(NBUF):
        pltpu.make_async_copy(g_hbm.at[srcbuf.at[k0 + b]], rows.at[b],
                              gsems[b]).wait()
        pltpu.async_copy(rows.at[b], acc.at[dstbuf.at[k0 + b]], ssems[b],
                         add=True)
    for b in range(NBUF):
        pltpu.make_async_copy(rows.at[b], acc.at[dstbuf.at[k0 + b]],
                              ssems[b]).wait()

    plsc.subcore_barrier()
    pltpu.sync_copy(acc.at[pl.ds(s * RPT, RPT)], out_hbm.at[c, pl.ds(s * RPT, RPT)])


def _dis_from(degp_ref):
    d = degp_ref[:, 0:1] + degp_ref[:, 1:2] + 1.0
    return lax.rsqrt(d)


def _tc_a_body(x_ref, w1_ref, degp_ref, g1_ref):
    dis = _dis_from(degp_ref)
    h = jnp.dot(x_ref[...], w1_ref[...], preferred_element_type=jnp.float32)
    g1_ref[...] = dis * h


def _tc_b_body(degp_ref, g1_ref, s1_ref, b1_ref, q_ref):
    dis = _dis_from(degp_ref)
    ssum = s1_ref[0, :N, :] + s1_ref[1, :N, :]
    out1 = dis * (ssum + g1_ref[...]) + b1_ref[...]
    q_ref[...] = dis * jnp.maximum(out1, 0.0)


def _tc_c_body(degp_ref, q_ref, r_ref, w2_ref, b2_ref, o_ref):
    dis = _dis_from(degp_ref)
    rsum = r_ref[0, :N, :] + r_ref[1, :N, :]
    t = dis * (rsum + q_ref[...])
    o = jnp.dot(t, w2_ref[...], preferred_element_type=jnp.float32) + b2_ref[...]
    a = o[:, 0:1]
    b = o[:, 1:2]
    m = jnp.maximum(a, b)
    lse = m + jnp.log(jnp.exp(a - m) + jnp.exp(b - m))
    o_ref[...] = o - lse


def _vmem_call(body, n_in, out_shape):
    return pl.pallas_call(
        body,
        out_shape=out_shape,
        in_specs=[pl.BlockSpec(memory_space=pltpu.VMEM)] * n_in,
        out_specs=pl.BlockSpec(memory_space=pltpu.VMEM),
    )


@jax.jit
def kernel(x, edge_index, W1, b1, W2, b2):
    src = edge_index[0].astype(jnp.int32)
    dst = edge_index[1].astype(jnp.int32)
    pad_iota = jnp.arange(E_PAD - E, dtype=jnp.int32)
    src_p = jnp.concatenate([src, pad_iota % N])
    dst_p = jnp.concatenate([dst, N + pad_iota % (N_PAD - N)])
    src_t = src_p.reshape(NW, CH, BATCH)
    dst_t = dst_p.reshape(NW, CH, BATCH)

    degp = _deg_kernel(dst_t)
    degp_t = degp.T[:N, :]

    g1 = _vmem_call(
        _tc_a_body, 3, jax.ShapeDtypeStruct((N, H), jnp.float32)
    )(x, W1, degp_t)

    s1 = _agg_kernel(g1, src_t, dst_t)

    q = _vmem_call(
        _tc_b_body, 4, jax.ShapeDtypeStruct((N, H), jnp.float32)
    )(degp_t, g1, s1, b1.reshape(1, H))

    r = _agg_kernel(q, src_t, dst_t)

    out = _vmem_call(
        _tc_c_body, 5, jax.ShapeDtypeStruct((N, OUT), jnp.float32)
    )(degp_t, q, r, W2, b2.reshape(1, OUT))
    return out
